# baseline (speedup 1.0000x reference)
import functools

import jax
import jax.numpy as jnp
from jax.experimental import pallas as pl
from jax.experimental.pallas import tpu as pltpu

_R = 256
_C = 1024
_S = 8
_THR = 0.5
_EPS = 1e-9
_BIG = 2.0e9


def _nms_kernel(cchunk, x1_ref, y1_ref, x2_ref, y2_ref, ar_ref, out_ref,
                px1_ref, py1_ref, px2_ref, py2_ref, par_ref,
                sx1_ref, sy1_ref, sx2_ref, sy2_ref, sar_ref):
    t = pl.program_id(0)
    base = t * _R

    @pl.when(t == 0)
    def _init():
        px1_ref[...] = jnp.full_like(px1_ref, _BIG)
        py1_ref[...] = jnp.full_like(py1_ref, _BIG)
        px2_ref[...] = jnp.full_like(px2_ref, _BIG)
        py2_ref[...] = jnp.full_like(py2_ref, _BIG)
        par_ref[...] = jnp.zeros_like(par_ref)

    xr1 = x1_ref[:, pl.ds(base, _R)].reshape(_R, 1)
    yr1 = y1_ref[:, pl.ds(base, _R)].reshape(_R, 1)
    xr2 = x2_ref[:, pl.ds(base, _R)].reshape(_R, 1)
    yr2 = y2_ref[:, pl.ds(base, _R)].reshape(_R, 1)
    arr = ar_ref[:, pl.ds(base, _R)].reshape(_R, 1)
    sx1_ref[...] = xr1
    sy1_ref[...] = yr1
    sx2_ref[...] = xr2
    sy2_ref[...] = yr2
    sar_ref[...] = arr

    def cross_body(c, sup):
        cb = c * cchunk
        xc1 = px1_ref[:, pl.ds(cb, cchunk)]
        yc1 = py1_ref[:, pl.ds(cb, cchunk)]
        xc2 = px2_ref[:, pl.ds(cb, cchunk)]
        yc2 = py2_ref[:, pl.ds(cb, cchunk)]
        arc = par_ref[:, pl.ds(cb, cchunk)]
        w = jnp.maximum(jnp.minimum(xr2, xc2) - jnp.maximum(xr1, xc1), 0.0)
        h = jnp.maximum(jnp.minimum(yr2, yc2) - jnp.maximum(yr1, yc1), 0.0)
        inter = w * h
        iou = inter / (arc + arr - inter + _EPS)
        row_max = jnp.max(iou, axis=1, keepdims=True).reshape(1, _R)
        return jnp.maximum(sup, row_max)

    nchunks = (t * _R + cchunk - 1) // cchunk
    sup = jax.lax.fori_loop(
        0, nchunks, cross_body, jnp.zeros((1, _R), jnp.float32), unroll=False
    )
    keep0 = jnp.where(sup > _THR, 0.0, 1.0)

    tx1 = x1_ref[:, pl.ds(base, _R)]
    ty1 = y1_ref[:, pl.ds(base, _R)]
    tx2 = x2_ref[:, pl.ds(base, _R)]
    ty2 = y2_ref[:, pl.ds(base, _R)]
    tar = ar_ref[:, pl.ds(base, _R)]
    j = jax.lax.broadcasted_iota(jnp.int32, (1, _R), 1)

    def slab_body(s, keep):
        k8 = s * _S
        rx1 = sx1_ref[pl.ds(k8, _S), :]
        ry1 = sy1_ref[pl.ds(k8, _S), :]
        rx2 = sx2_ref[pl.ds(k8, _S), :]
        ry2 = sy2_ref[pl.ds(k8, _S), :]
        rar = sar_ref[pl.ds(k8, _S), :]
        w = jnp.maximum(jnp.minimum(rx2, tx2) - jnp.maximum(rx1, tx1), 0.0)
        h = jnp.maximum(jnp.minimum(ry2, ty2) - jnp.maximum(ry1, ty1), 0.0)
        inter = w * h
        iou8 = inter / (rar + tar - inter + _EPS)

        for r in range(_S):
            k = k8 + r
            row = iou8[r:r + 1, :]
            kk = jnp.max(jnp.where(j == k, keep, 0.0), axis=1, keepdims=True)
            suppress = (row > _THR) & (j > k) & (kk > 0.0)
            keep = jnp.where(suppress, 0.0, keep)
        return keep

    keep = jax.lax.fori_loop(0, _R // _S, slab_body, keep0, unroll=False)

    out_ref[:, pl.ds(base, _R)] = keep
    kept = keep > 0.0
    px1_ref[:, pl.ds(base, _R)] = jnp.where(kept, tx1, _BIG)
    py1_ref[:, pl.ds(base, _R)] = jnp.where(kept, ty1, _BIG)
    px2_ref[:, pl.ds(base, _R)] = jnp.where(kept, tx2, _BIG)
    py2_ref[:, pl.ds(base, _R)] = jnp.where(kept, ty2, _BIG)
    par_ref[:, pl.ds(base, _R)] = jnp.where(kept, tar, 0.0)


def _run_nms(x1, y1, x2, y2, area, interpret=False):
    npad = x1.shape[1]
    cchunk = min(_C, npad)
    spec = pl.BlockSpec((1, npad), lambda t: (0, 0))
    big = pltpu.VMEM((1, npad), jnp.float32)
    small = pltpu.VMEM((_R, 1), jnp.float32)
    keep = pl.pallas_call(
        functools.partial(_nms_kernel, cchunk),
        grid=(npad // _R,),
        in_specs=[spec] * 5,
        out_specs=spec,
        out_shape=jax.ShapeDtypeStruct((1, npad), jnp.float32),
        scratch_shapes=[big] * 5 + [small] * 5,
        interpret=interpret,
    )(x1, y1, x2, y2, area)
    return keep[0]


def kernel(boxes, scores):
    n = scores.shape[0]
    order = jnp.argsort(-scores)
    b = boxes[order]
    blk = max(_R, _C)
    npad = ((n + blk - 1) // blk) * blk
    pad = npad - n
    bp = jnp.concatenate([b, jnp.full((pad, 4), _BIG, jnp.float32)], axis=0)
    x1 = bp[:, 0].reshape(1, npad)
    y1 = bp[:, 1].reshape(1, npad)
    x2 = bp[:, 2].reshape(1, npad)
    y2 = bp[:, 3].reshape(1, npad)
    area = (x2 - x1) * (y2 - y1)
    keep_sorted = _run_nms(x1, y1, x2, y2, area)[:n] > 0.0
    kept = jnp.zeros((n,), dtype=bool).at[order].set(keep_sorted)
    return jnp.where(kept, scores, 0.0)


# speedup vs baseline: 1.0582x; 1.0582x over previous
import functools

import jax
import jax.numpy as jnp
from jax.experimental import pallas as pl
from jax.experimental.pallas import tpu as pltpu

_R = 256
_C = 1024
_S = 8
_THR = 0.5
_EPS = 1e-9
_BIG = 2.0e9


def _nms_kernel(cchunk, x1_ref, y1_ref, x2_ref, y2_ref, ar_ref, out_ref,
                px1_ref, py1_ref, px2_ref, py2_ref, par_ref,
                sx1_ref, sy1_ref, sx2_ref, sy2_ref, sar_ref):
    t = pl.program_id(0)
    base = t * _R

    @pl.when(t == 0)
    def _init():
        px1_ref[...] = jnp.full_like(px1_ref, _BIG)
        py1_ref[...] = jnp.full_like(py1_ref, _BIG)
        px2_ref[...] = jnp.full_like(px2_ref, _BIG)
        py2_ref[...] = jnp.full_like(py2_ref, _BIG)
        par_ref[...] = jnp.zeros_like(par_ref)

    xr1 = x1_ref[:, pl.ds(base, _R)].reshape(_R, 1)
    yr1 = y1_ref[:, pl.ds(base, _R)].reshape(_R, 1)
    xr2 = x2_ref[:, pl.ds(base, _R)].reshape(_R, 1)
    yr2 = y2_ref[:, pl.ds(base, _R)].reshape(_R, 1)
    arr = ar_ref[:, pl.ds(base, _R)].reshape(_R, 1)
    sx1_ref[...] = xr1
    sy1_ref[...] = yr1
    sx2_ref[...] = xr2
    sy2_ref[...] = yr2
    sar_ref[...] = arr

    def cross_body(c, sup):
        cb = c * cchunk
        xc1 = px1_ref[:, pl.ds(cb, cchunk)]
        yc1 = py1_ref[:, pl.ds(cb, cchunk)]
        xc2 = px2_ref[:, pl.ds(cb, cchunk)]
        yc2 = py2_ref[:, pl.ds(cb, cchunk)]
        arc = par_ref[:, pl.ds(cb, cchunk)]
        w = jnp.maximum(jnp.minimum(xr2, xc2) - jnp.maximum(xr1, xc1), 0.0)
        h = jnp.maximum(jnp.minimum(yr2, yc2) - jnp.maximum(yr1, yc1), 0.0)
        inter = w * h
        iou = inter / (arc + arr - inter + _EPS)
        row_max = jnp.max(iou, axis=1, keepdims=True).reshape(1, _R)
        return jnp.maximum(sup, row_max)

    nchunks = (t * _R + cchunk - 1) // cchunk
    sup = jax.lax.fori_loop(
        0, nchunks, cross_body, jnp.zeros((1, _R), jnp.float32), unroll=False
    )
    keep0 = jnp.where(sup > _THR, 0.0, 1.0)

    tx1 = x1_ref[:, pl.ds(base, _R)]
    ty1 = y1_ref[:, pl.ds(base, _R)]
    tx2 = x2_ref[:, pl.ds(base, _R)]
    ty2 = y2_ref[:, pl.ds(base, _R)]
    tar = ar_ref[:, pl.ds(base, _R)]
    j = jax.lax.broadcasted_iota(jnp.int32, (1, _R), 1)

    l8 = jax.lax.broadcasted_iota(jnp.int32, (1, _S), 1)
    i8 = jax.lax.broadcasted_iota(jnp.int32, (_S, 1), 0)

    def slab_body(s, keep):
        k8 = s * _S
        rx1 = sx1_ref[pl.ds(k8, _S), :]
        ry1 = sy1_ref[pl.ds(k8, _S), :]
        rx2 = sx2_ref[pl.ds(k8, _S), :]
        ry2 = sy2_ref[pl.ds(k8, _S), :]
        rar = sar_ref[pl.ds(k8, _S), :]
        w = jnp.maximum(jnp.minimum(rx2, tx2) - jnp.maximum(rx1, tx1), 0.0)
        h = jnp.maximum(jnp.minimum(ry2, ty2) - jnp.maximum(ry1, ty1), 0.0)
        inter = w * h
        iou8 = inter / (rar + tar - inter + _EPS)

        cx1 = rx1.reshape(1, _S)
        cy1 = ry1.reshape(1, _S)
        cx2 = rx2.reshape(1, _S)
        cy2 = ry2.reshape(1, _S)
        car = rar.reshape(1, _S)
        cw = jnp.maximum(jnp.minimum(rx2, cx2) - jnp.maximum(rx1, cx1), 0.0)
        ch = jnp.maximum(jnp.minimum(ry2, cy2) - jnp.maximum(ry1, cy1), 0.0)
        cinter = cw * ch
        ciou = cinter / (rar + car - cinter + _EPS)
        chit = jnp.where((ciou > _THR) & (l8 > i8), 1.0, 0.0)

        g = [jnp.max(jnp.where(j == k8 + r, keep, 0.0), axis=1, keepdims=True)
             for r in range(_S)]
        for r in range(1, _S):
            acc = g[r]
            for q in range(r):
                acc = acc * (1.0 - chit[q:q + 1, r:r + 1] * g[q])
            g[r] = acc
        for r in range(_S):
            suppress = (iou8[r:r + 1, :] > _THR) & (j > k8 + r) & (g[r] > 0.0)
            keep = jnp.where(suppress, 0.0, keep)
        return keep

    keep = jax.lax.fori_loop(0, _R // _S, slab_body, keep0, unroll=False)

    out_ref[:, pl.ds(base, _R)] = keep
    kept = keep > 0.0
    px1_ref[:, pl.ds(base, _R)] = jnp.where(kept, tx1, _BIG)
    py1_ref[:, pl.ds(base, _R)] = jnp.where(kept, ty1, _BIG)
    px2_ref[:, pl.ds(base, _R)] = jnp.where(kept, tx2, _BIG)
    py2_ref[:, pl.ds(base, _R)] = jnp.where(kept, ty2, _BIG)
    par_ref[:, pl.ds(base, _R)] = jnp.where(kept, tar, 0.0)


def _run_nms(x1, y1, x2, y2, area, interpret=False):
    npad = x1.shape[1]
    cchunk = min(_C, npad)
    spec = pl.BlockSpec((1, npad), lambda t: (0, 0))
    big = pltpu.VMEM((1, npad), jnp.float32)
    small = pltpu.VMEM((_R, 1), jnp.float32)
    keep = pl.pallas_call(
        functools.partial(_nms_kernel, cchunk),
        grid=(npad // _R,),
        in_specs=[spec] * 5,
        out_specs=spec,
        out_shape=jax.ShapeDtypeStruct((1, npad), jnp.float32),
        scratch_shapes=[big] * 5 + [small] * 5,
        interpret=interpret,
    )(x1, y1, x2, y2, area)
    return keep[0]


def kernel(boxes, scores):
    n = scores.shape[0]
    order = jnp.argsort(-scores)
    b = boxes[order]
    blk = max(_R, _C)
    npad = ((n + blk - 1) // blk) * blk
    pad = npad - n
    bp = jnp.concatenate([b, jnp.full((pad, 4), _BIG, jnp.float32)], axis=0)
    x1 = bp[:, 0].reshape(1, npad)
    y1 = bp[:, 1].reshape(1, npad)
    x2 = bp[:, 2].reshape(1, npad)
    y2 = bp[:, 3].reshape(1, npad)
    area = (x2 - x1) * (y2 - y1)
    keep_sorted = _run_nms(x1, y1, x2, y2, area)[:n] > 0.0
    kept = jnp.zeros((n,), dtype=bool).at[order].set(keep_sorted)
    return jnp.where(kept, scores, 0.0)
